# Initial kernel scaffold; baseline (speedup 1.0000x reference)
import jax
import jax.numpy as jnp
from jax.experimental import pallas as pl


def kernel(x, voxel_grid):
    raise NotImplementedError("write your pallas kernel here")


# speedup vs baseline: 2.6580x; 2.6580x over previous
import functools

import jax
import jax.numpy as jnp
from jax import lax
from jax.experimental import pallas as pl
from jax.experimental.pallas import tpu as pltpu
from jax.experimental.pallas import tpu_sc as plsc

WIDTH = 32
RES = 128
SCALE = 1.5
N_PTS = 1048576

NC = 2
NS = 16
LANES = 16
NW = NC * NS
PER_W = N_PTS // NW
CHUNK = 128
NCHUNK = PER_W // CHUNK


def _sc_body(x_hbm, tab_hbm, out_hbm, xv, wqv, idxv, rowsv, outv, sem):
    cid = lax.axis_index("c")
    sid = lax.axis_index("s")
    wid = sid * NC + cid
    base = wid * PER_W
    lane = lax.iota(jnp.int32, LANES)

    def chunk_body(t, carry):
        pt = base + t * CHUNK
        pltpu.sync_copy(x_hbm.at[pl.ds(pt * 3, CHUNK * 3)], xv)

        for j in range(CHUNK // LANES):
            row = lane + j * LANES
            r3 = row * 3
            px = plsc.load_gather(xv, [r3])
            py = plsc.load_gather(xv, [r3 + 1])
            pz = plsc.load_gather(xv, [r3 + 2])
            m = ((jnp.abs(px) < SCALE) & (jnp.abs(py) < SCALE)
                 & (jnp.abs(pz) < SCALE))
            mf = jnp.where(m, 1.0, 0.0).astype(jnp.float32)
            gx = (jnp.clip(px / SCALE, -1.0, 1.0) + 1.0) * 0.5 * (RES - 1)
            gy = (jnp.clip(py / SCALE, -1.0, 1.0) + 1.0) * 0.5 * (RES - 1)
            gz = (jnp.clip(pz / SCALE, -1.0, 1.0) + 1.0) * 0.5 * (RES - 1)
            x0 = gx.astype(jnp.int32)
            y0 = gy.astype(jnp.int32)
            z0 = gz.astype(jnp.int32)
            wx = gx - x0.astype(jnp.float32)
            wy = gy - y0.astype(jnp.float32)
            wz = gz - z0.astype(jnp.float32)
            x1 = jnp.minimum(x0 + 1, RES - 1)
            y1 = jnp.minimum(y0 + 1, RES - 1)
            z1 = jnp.minimum(z0 + 1, RES - 1)
            zy00 = z0 * (RES * RES) + y0 * RES
            zy01 = z0 * (RES * RES) + y1 * RES
            zy10 = z1 * (RES * RES) + y0 * RES
            zy11 = z1 * (RES * RES) + y1 * RES
            sl = pl.ds(j * LANES, LANES)
            idxv[0, sl] = zy00 + x0
            idxv[1, sl] = zy00 + x1
            idxv[2, sl] = zy01 + x0
            idxv[3, sl] = zy01 + x1
            idxv[4, sl] = zy10 + x0
            idxv[5, sl] = zy10 + x1
            idxv[6, sl] = zy11 + x0
            idxv[7, sl] = zy11 + x1
            q = (row * 4).astype(jnp.int32)
            plsc.store_scatter(wqv, [q], wx)
            plsc.store_scatter(wqv, [q + 1], wy)
            plsc.store_scatter(wqv, [q + 2], (1.0 - wz) * mf)
            plsc.store_scatter(wqv, [q + 3], wz * mf)

        cps = [pltpu.async_copy(tab_hbm.at[idxv.at[c]], rowsv.at[c], sem)
               for c in range(8)]
        for cp in cps:
            cp.wait()

        def pt_body(i, c):
            wvec = wqv[pl.ds(i * 4, LANES)]
            wx = wvec[0]
            wy = wvec[1]
            w0 = wvec[2]
            w1 = wvec[3]
            for h in range(WIDTH // LANES):
                hs = pl.ds(h * LANES, LANES)
                r0 = rowsv[0, i, hs]
                r1 = rowsv[1, i, hs]
                r2 = rowsv[2, i, hs]
                r3 = rowsv[3, i, hs]
                r4 = rowsv[4, i, hs]
                r5 = rowsv[5, i, hs]
                r6 = rowsv[6, i, hs]
                r7 = rowsv[7, i, hs]
                a0 = r0 + wx * (r1 - r0)
                a1 = r2 + wx * (r3 - r2)
                a2 = r4 + wx * (r5 - r4)
                a3 = r6 + wx * (r7 - r6)
                b0 = a0 + wy * (a1 - a0)
                b1 = a2 + wy * (a3 - a2)
                outv[i, hs] = b0 * w0 + b1 * w1
            return c

        lax.fori_loop(0, CHUNK, pt_body, 0)
        pltpu.sync_copy(outv, out_hbm.at[pl.ds(pt, CHUNK), :])
        return carry

    lax.fori_loop(0, NCHUNK, chunk_body, 0)


@jax.jit
def _run(x, table):
    mesh = plsc.VectorSubcoreMesh(core_axis_name="c", subcore_axis_name="s",
                                  num_cores=NC, num_subcores=NS)
    f = pl.kernel(
        _sc_body,
        out_type=jax.ShapeDtypeStruct((N_PTS, WIDTH), jnp.float32),
        mesh=mesh,
        compiler_params=pltpu.CompilerParams(needs_layout_passes=False,
                                             use_tc_tiling_on_sc=False),
        scratch_types=[
            pltpu.VMEM((CHUNK * 3,), jnp.float32),
            pltpu.VMEM((CHUNK * 4 + LANES,), jnp.float32),
            pltpu.VMEM((8, CHUNK), jnp.int32),
            pltpu.VMEM((8, CHUNK, WIDTH), jnp.float32),
            pltpu.VMEM((CHUNK, WIDTH), jnp.float32),
            pltpu.SemaphoreType.DMA,
        ],
    )
    return f(x, table)


def kernel(x, voxel_grid):
    table = voxel_grid[0].transpose(1, 2, 3, 0).reshape(RES * RES * RES, WIDTH)
    return _run(x.reshape(N_PTS * 3), table)
